# baseline (speedup 1.0000x reference)
import functools

import jax
import jax.numpy as jnp
from jax import lax
from jax.experimental import pallas as pl
from jax.experimental.pallas import tpu as pltpu
from jax.experimental.pallas import tpu_sc as plsc

L = 16


def _build_sc_kernel(R, C, K_pad):
    NW = 32
    rows_per_w = R // NW
    chunk_rows = 16
    chunk = chunk_rows * C
    nchunk = rows_per_w // chunk_rows
    period = 2 * C
    nt = period // L
    rep = chunk // period

    mesh = plsc.VectorSubcoreMesh(core_axis_name="c", subcore_axis_name="s")

    @functools.partial(
        pl.kernel,
        mesh=mesh,
        compiler_params=pltpu.CompilerParams(needs_layout_passes=False),
        out_type=jax.ShapeDtypeStruct((R * C,), jnp.float32),
        scratch_types=[
            pltpu.VMEM((2 * L,), jnp.float32),
            pltpu.VMEM((K_pad,), jnp.int32),
            pltpu.VMEM((period,), jnp.float32),
            pltpu.VMEM((period,), jnp.float32),
            pltpu.VMEM((chunk,), jnp.float32),
            pltpu.VMEM((chunk,), jnp.float32),
            pltpu.VMEM((chunk,), jnp.float32),
            pltpu.VMEM((chunk,), jnp.float32),
            pltpu.SemaphoreType.DMA,
            pltpu.SemaphoreType.DMA,
            pltpu.SemaphoreType.DMA,
            pltpu.SemaphoreType.DMA,
        ],
    )
    def sc_kernel(x_hbm, ab_hbm, clss_hbm, out_hbm,
                  ab_v, clss_v, a_v, b_v, in0, in1, out0, out1,
                  isem0, isem1, osem0, osem1):
        wid = lax.axis_index("s") * 2 + lax.axis_index("c")
        base = wid * (rows_per_w * C)

        ins = [in0, in1]
        outs = [out0, out1]
        isems = [isem0, isem1]
        osems = [osem0, osem1]

        in_copies = {}
        for c in range(min(2, nchunk)):
            in_copies[c] = pltpu.async_copy(
                x_hbm.at[pl.ds(base + c * chunk, chunk)], ins[c % 2],
                isems[c % 2])

        pltpu.sync_copy(ab_hbm, ab_v)
        pltpu.sync_copy(clss_hbm, clss_v)

        ones = jnp.full((L,), 1.0, jnp.float32)

        def init_body(i, _):
            a_v[pl.ds(i * L, L)] = ones
            b_v[pl.ds(i * L, L)] = ones
            return 0

        lax.fori_loop(0, nt, init_body, 0)

        alpha_vec = ab_v[pl.ds(0, L)]
        beta_vec = ab_v[pl.ds(L, L)]
        shift = jnp.full((L,), C, jnp.int32)
        for k in range(K_pad // L):
            idx = clss_v[pl.ds(k * L, L)]
            plsc.store_scatter(a_v, [idx], alpha_vec)
            plsc.store_scatter(a_v, [idx + shift], alpha_vec)
            plsc.store_scatter(b_v, [idx], beta_vec)
            plsc.store_scatter(b_v, [idx + shift], beta_vec)

        out_copies = {}
        for c in range(nchunk):
            b = c % 2
            in_copies[c].wait()
            if c >= 2:
                out_copies[c - 2].wait()

            def chunk_body(t, _, b=b):
                av = a_v[pl.ds(t * L, L)]
                bv = b_v[pl.ds(t * L, L)]
                for p in range(rep):
                    off = t * L + p * period
                    outs[b][pl.ds(off, L)] = av * ins[b][pl.ds(off, L)] + bv
                return 0

            lax.fori_loop(0, nt, chunk_body, 0)

            out_copies[c] = pltpu.async_copy(
                outs[b], out_hbm.at[pl.ds(base + c * chunk, chunk)], osems[b])
            if c + 2 < nchunk:
                in_copies[c + 2] = pltpu.async_copy(
                    x_hbm.at[pl.ds(base + (c + 2) * chunk, chunk)], ins[b],
                    isems[b])

        for c in range(max(0, nchunk - 2), nchunk):
            out_copies[c].wait()

    return sc_kernel


def kernel(x, alpha, beta, clss):
    R, C = x.shape
    K = clss.shape[0]
    K_pad = -(-K // L) * L
    assert R % 32 == 0 and (2 * C) % L == 0

    ab = jnp.concatenate([
        jnp.broadcast_to(alpha.astype(jnp.float32), (L,)),
        jnp.broadcast_to(beta.astype(jnp.float32), (L,)),
    ])
    clss_pad = jnp.concatenate(
        [clss, jnp.broadcast_to(clss[:1], (K_pad - K,))]).astype(jnp.int32)

    sc = _build_sc_kernel(R, C, K_pad)
    return sc(x.reshape(-1), ab, clss_pad).reshape(R, C)
